# baseline (speedup 1.0000x reference)
import functools

import jax
import jax.numpy as jnp
from jax import lax
from jax.experimental import pallas as pl
from jax.experimental.pallas import tpu as pltpu
from jax.experimental.pallas import tpu_sc as plsc

RATIO = 0.5
NC, NS = 2, 16
NW = NC * NS
C_EDGE = 128
NB = 16
F32 = jnp.float32
HI = lax.Precision.HIGHEST


def _sc_segsum(D, KI, PN, NH=1):
    rps = PN // NS
    NGH = KI // NH
    assert NGH * NH == KI and NGH % 2 == 0
    mesh = plsc.VectorSubcoreMesh(
        core_axis_name="c", subcore_axis_name="s", num_cores=NC, num_subcores=NS)

    @functools.partial(
        pl.kernel,
        out_type=jax.ShapeDtypeStruct((NC, PN, D), F32),
        mesh=mesh,
        compiler_params=pltpu.CompilerParams(use_tc_tiling_on_sc=False),
        scratch_types=[
            pltpu.VMEM((NGH, C_EDGE), jnp.int32),
            pltpu.VMEM((NGH, C_EDGE), jnp.int32),
            pltpu.VMEM((C_EDGE, D), F32),
            pltpu.VMEM((C_EDGE, D), F32),
            pltpu.VMEM_SHARED((PN, D), F32),
            pltpu.SemaphoreType.DMA,
            pltpu.SemaphoreType.DMA,
        ],
    )
    def k(table, srcp, dstp, zeros_h, out, srcv, dstv, rows0, rows1, acc,
          sem0, sem1):
        cid = lax.axis_index("c")
        sid = lax.axis_index("s")
        wid = sid * NC + cid
        pltpu.sync_copy(zeros_h.at[pl.ds(sid * rps, rps)],
                        acc.at[pl.ds(sid * rps, rps)])
        plsc.subcore_barrier()

        def body(p, carry):
            i = 2 * p
            pltpu.async_copy(table.at[srcv.at[i + 1]], rows1, sem1)
            pltpu.make_async_copy(table.at[pl.ds(0, C_EDGE)], rows0, sem0).wait()
            pltpu.sync_copy(rows0, acc.at[dstv.at[i]], add=True)

            @pl.when(i + 2 < NGH)
            def _():
                pltpu.async_copy(table.at[srcv.at[i + 2]], rows0, sem0)

            pltpu.make_async_copy(table.at[pl.ds(0, C_EDGE)], rows1, sem1).wait()
            pltpu.sync_copy(rows1, acc.at[dstv.at[i + 1]], add=True)
            return carry

        for hh in range(NH):
            pltpu.sync_copy(srcp.at[wid, pl.ds(hh * NGH, NGH)], srcv)
            pltpu.sync_copy(dstp.at[wid, pl.ds(hh * NGH, NGH)], dstv)
            pltpu.async_copy(table.at[srcv.at[0]], rows0, sem0)
            lax.fori_loop(0, NGH // 2, body, 0)

        plsc.subcore_barrier()
        pltpu.sync_copy(acc.at[pl.ds(sid * rps, rps)],
                        out.at[cid, pl.ds(sid * rps, rps)])

    return k


def _sc_histogram(KI, PN):
    D = NB
    rps = PN // NS
    mesh = plsc.VectorSubcoreMesh(
        core_axis_name="c", subcore_axis_name="s", num_cores=NC, num_subcores=NS)

    @functools.partial(
        pl.kernel,
        out_type=jax.ShapeDtypeStruct((NC, PN, D), F32),
        mesh=mesh,
        compiler_params=pltpu.CompilerParams(use_tc_tiling_on_sc=False),
        scratch_types=[
            pltpu.VMEM((KI, C_EDGE), jnp.int32),
            pltpu.VMEM((C_EDGE, D), F32),
            pltpu.VMEM_SHARED((PN, D), F32),
        ],
    )
    def k(ones_h, dstp, zeros_h, out, dstv, rows, acc):
        cid = lax.axis_index("c")
        sid = lax.axis_index("s")
        wid = sid * NC + cid
        pltpu.sync_copy(zeros_h.at[pl.ds(sid * rps, rps)],
                        acc.at[pl.ds(sid * rps, rps)])
        pltpu.sync_copy(dstp.at[wid], dstv)
        pltpu.sync_copy(ones_h.at[pl.ds(0, C_EDGE)], rows)
        plsc.subcore_barrier()

        def body(i, carry):
            pltpu.sync_copy(rows, acc.at[dstv.at[i]], add=True)
            return carry

        lax.fori_loop(0, KI, body, 0)
        plsc.subcore_barrier()
        pltpu.sync_copy(acc.at[pl.ds(sid * rps, rps)],
                        out.at[cid, pl.ds(sid * rps, rps)])

    return k


def _prep_body(x_ref, w_ref, dinv_ref, hs_ref):
    dinv = dinv_ref[...][:, :1]
    hs_ref[...] = jnp.dot(x_ref[...], w_ref[...], preferred_element_type=F32) * dinv


def _post1_body(pp_ref, hs_ref, dinv_ref, b_ref, wsc_ref,
                h_ref, srel_ref, sroot_ref):
    dinv = dinv_ref[...][:, :1]
    h = jnp.maximum(dinv * (pp_ref[0] + pp_ref[1] + hs_ref[...]) + b_ref[...], 0.0)
    h_ref[...] = h
    sb = jnp.dot(h, wsc_ref[...], preferred_element_type=F32)
    srel_ref[...] = jnp.broadcast_to(sb[:, :1], srel_ref.shape)
    sroot_ref[...] = jnp.broadcast_to(sb[:, 1:2], sroot_ref.shape)


def _post2_body(pp_ref, hs_ref, dinv_ref, b_ref, wsc_ref, kept_ref,
                h_ref, srel_ref, sroot_ref):
    kept = kept_ref[...][:, :1]
    dinv = dinv_ref[...][:, :1]
    h = jnp.maximum(dinv * (pp_ref[0] + pp_ref[1] + hs_ref[...]) + b_ref[...], 0.0)
    h = jnp.where(kept > 0, h, 0.0)
    h_ref[...] = h
    sb = jnp.dot(h, wsc_ref[...], preferred_element_type=F32)
    srel_ref[...] = jnp.broadcast_to(sb[:, :1], srel_ref.shape)
    sroot_ref[...] = jnp.broadcast_to(sb[:, 1:2], sroot_ref.shape)


def _make_sel_body(PN, with_elig):
    def select(scoreT_ref, batchT_ref, elig_ref, keptT_ref, cnt_ref):
        onehot = batchT_ref[...] == lax.broadcasted_iota(jnp.int32, (NB, PN), 0)
        if with_elig:
            onehot = onehot & (elig_ref[...] > 0)
        score = scoreT_ref[...]
        cnt = jnp.sum(onehot.astype(F32), axis=1, keepdims=True)
        kk = jnp.ceil(RATIO * cnt)
        u = lax.bitcast_convert_type(score, jnp.uint32)
        flip = jnp.where(u >> jnp.uint32(31) == jnp.uint32(1),
                         jnp.uint32(0xFFFFFFFF), jnp.uint32(0x80000000))
        key = u ^ flip

        def rbody(j, t):
            cand = t | jnp.left_shift(jnp.uint32(1), (31 - j).astype(jnp.uint32))
            c = jnp.sum(((key >= cand) & onehot).astype(F32), axis=1,
                        keepdims=True)
            return jnp.where(c >= kk, cand, t)

        V = lax.fori_loop(0, 32, rbody, jnp.zeros((NB, 1), jnp.uint32))
        gt = (key > V) & onehot
        eq = (key == V) & onehot
        m = kk - jnp.sum(gt.astype(F32), axis=1, keepdims=True)
        ridx = lax.broadcasted_iota(jnp.int32, (NB, PN), 1)

        def bbody(j, lohi):
            lo, hi = lohi
            mid = (lo + hi) // 2
            c = jnp.sum((eq & (ridx <= mid)).astype(F32), axis=1, keepdims=True)
            hit = c >= m
            return jnp.where(hit, lo, mid + 1), jnp.where(hit, mid, hi)

        _, J = lax.fori_loop(0, 14, bbody,
                             (jnp.zeros((NB, 1), jnp.int32),
                              jnp.full((NB, 1), PN - 1, jnp.int32)))
        kept = (gt | (eq & (ridx <= J))) & (kk >= 1.0)
        keptT_ref[...] = kept.astype(F32)
        ck = jnp.sum(kept.astype(F32), axis=1, keepdims=True)
        cnt_ref[...] = jnp.broadcast_to(ck, (NB, 128))

    if with_elig:
        def body(scoreT_ref, batchT_ref, elig_ref, keptT_ref, cnt_ref):
            select(scoreT_ref, batchT_ref, elig_ref, keptT_ref, cnt_ref)
    else:
        def body(scoreT_ref, batchT_ref, keptT_ref, cnt_ref):
            select(scoreT_ref, batchT_ref, None, keptT_ref, cnt_ref)
    return body


def _gate_pool_body(h_ref, gate_ref, keptn_ref,
                    h1_ref, kept16_ref, sums_ref, mx_ref):
    i = pl.program_id(0)
    kept_row = jnp.sum(keptn_ref[...], axis=1, keepdims=True)
    hg = h_ref[...] * (gate_ref[...][:, :1] * kept_row)
    h1_ref[...] = hg
    kept16_ref[...] = jnp.broadcast_to(kept_row, kept16_ref.shape)
    part = lax.dot_general(keptn_ref[...], hg, (((0,), (0,)), ((), ())),
                           preferred_element_type=F32)
    neg = jnp.float32(-jnp.inf)
    rows = []
    for b in range(NB):
        maskb = keptn_ref[...][:, b:b + 1]
        rows.append(jnp.max(jnp.where(maskb > 0, hg, neg), axis=0, keepdims=True))
    tile_mx = jnp.concatenate(rows, axis=0)

    @pl.when(i == 0)
    def _():
        sums_ref[...] = part
        mx_ref[...] = tile_mx

    @pl.when(i > 0)
    def _():
        sums_ref[...] = sums_ref[...] + part
        mx_ref[...] = jnp.maximum(mx_ref[...], tile_mx)


def _pool2_body(h_ref, gate_ref, keptn_ref, sums_ref, mx_ref):
    i = pl.program_id(0)
    kept_row = jnp.sum(keptn_ref[...], axis=1, keepdims=True)
    hg = h_ref[...] * (gate_ref[...][:, :1] * kept_row)
    part = lax.dot_general(keptn_ref[...], hg, (((0,), (0,)), ((), ())),
                           preferred_element_type=F32)
    neg = jnp.float32(-jnp.inf)
    rows = []
    for b in range(NB):
        maskb = keptn_ref[...][:, b:b + 1]
        rows.append(jnp.max(jnp.where(maskb > 0, hg, neg), axis=0, keepdims=True))
    tile_mx = jnp.concatenate(rows, axis=0)

    @pl.when(i == 0)
    def _():
        sums_ref[...] = part
        mx_ref[...] = tile_mx

    @pl.when(i > 0)
    def _():
        sums_ref[...] = sums_ref[...] + part
        mx_ref[...] = jnp.maximum(mx_ref[...], tile_mx)


def _head_body(sums1_ref, mx1_ref, cnt1_ref, sums2_ref, mx2_ref, cnt2_ref,
               w1a_ref, w1b_ref, b1_ref, w2_ref, b2_ref, w3_ref, b3_ref,
               logits_ref, probs_ref):
    c1 = cnt1_ref[...][:, :1]
    c2 = cnt2_ref[...][:, :1]
    mean1 = sums1_ref[...] / jnp.maximum(c1, 1.0)
    mean2 = sums2_ref[...] / jnp.maximum(c2, 1.0)
    mx1 = jnp.where(c1 > 0, mx1_ref[...], 0.0)
    mx2 = jnp.where(c2 > 0, mx2_ref[...], 0.0)
    zm = mean1 + mean2
    zx = mx1 + mx2
    z = jnp.maximum(
        jnp.dot(zm, w1a_ref[...], preferred_element_type=F32)
        + jnp.dot(zx, w1b_ref[...], preferred_element_type=F32)
        + b1_ref[...], 0.0)
    z = jnp.maximum(jnp.dot(z, w2_ref[...], preferred_element_type=F32,
                            precision=HI) + b2_ref[...], 0.0)
    logits = jnp.dot(z, w3_ref[...], preferred_element_type=F32,
                     precision=HI) + b3_ref[...]
    logits_ref[...] = logits
    mmax = jnp.max(logits, axis=1, keepdims=True)
    e = jnp.exp(logits - mmax)
    probs_ref[...] = e / jnp.sum(e, axis=1, keepdims=True)


def kernel(x, edge_index, batch, label, W1, b1, Ws1_rel, bs1, Ws1_root, W2, b2,
           Ws2_rel, bs2, Ws2_root, lin1_W, lin1_b, lin2_W, lin2_b, lin3_W, lin3_b):
    N = x.shape[0]
    E = edge_index.shape[1]
    H = W1.shape[1]
    PN = ((N + 511) // 512) * 512
    KI = -(-E // (NW * C_EDGE))
    KI = KI + (KI % 2)
    Rb = 1024 if PN % 1024 == 0 else 512

    src = edge_index[0].astype(jnp.int32)
    dst = edge_index[1].astype(jnp.int32)
    pad_e = NW * KI * C_EDGE - E
    pad_dst = N + (jnp.arange(pad_e, dtype=jnp.int32) % (PN - N))
    srcp = jnp.concatenate([src, jnp.zeros((pad_e,), jnp.int32)]).reshape(NW, KI, C_EDGE)
    dstp = jnp.concatenate([dst, pad_dst]).reshape(NW, KI, C_EDGE)
    xp = jnp.pad(x, ((0, PN - N), (0, 0)))
    batch_pad = jnp.pad(batch.astype(jnp.int32), (0, PN - N), constant_values=NB)
    batchT = jnp.broadcast_to(batch_pad[None, :], (NB, PN))
    ones16 = jnp.ones((PN, NB), F32)
    z16 = jnp.zeros((PN, NB), F32)
    z128 = jnp.zeros((PN, H), F32)
    wsc1 = jnp.concatenate([Ws1_rel, Ws1_root], axis=1)
    wsc2 = jnp.concatenate([Ws2_rel, Ws2_root], axis=1)

    seg16 = _sc_segsum(NB, KI, PN)
    seg128 = _sc_segsum(H, KI, PN, NH=2)

    row128 = pl.BlockSpec((Rb, H), lambda i: (i, 0))
    row16 = pl.BlockSpec((Rb, NB), lambda i: (i, 0))
    wspec = pl.BlockSpec((H, H), lambda i: (0, 0))
    w2spec = pl.BlockSpec((H, 2), lambda i: (0, 0))
    bspec = pl.BlockSpec((1, H), lambda i: (0, 0))
    pp128 = pl.BlockSpec((2, Rb, H), lambda i: (0, i, 0))
    accspec = pl.BlockSpec((NB, H), lambda i: (0, 0))
    sds = jax.ShapeDtypeStruct
    grid = PN // Rb

    def bcast16(v):
        return jnp.broadcast_to(v[:, None], (PN, NB))

    degp1 = _sc_histogram(KI, PN)(ones16, dstp, z16)
    deg1 = degp1[0, :, 0] + degp1[1, :, 0] + 1.0
    dinv1 = jnp.where(deg1 > 0, deg1 ** -0.5, 0.0)
    dinv116 = bcast16(dinv1)
    hs1 = pl.pallas_call(
        _prep_body, grid=(grid,), in_specs=[row128, wspec, row16],
        out_specs=row128, out_shape=sds((PN, H), F32))(xp, W1, dinv116)
    featp1 = seg128(hs1, srcp, dstp, z128)
    h, srel16, sroot16 = pl.pallas_call(
        _post1_body, grid=(grid,),
        in_specs=[pp128, row128, row16, bspec, w2spec],
        out_specs=[row128, row16, row16],
        out_shape=[sds((PN, H), F32), sds((PN, NB), F32), sds((PN, NB), F32)])(
        featp1, hs1, dinv116, b1.reshape(1, H), wsc1)
    aggp1 = seg16(srel16, srcp, dstp, z16)

    score1 = aggp1[0, :, 0] + aggp1[1, :, 0] + bs1[0] + sroot16[:, 0]
    scoreT1 = jnp.broadcast_to(score1[None, :], (NB, PN))
    keptT1, cnt1 = pl.pallas_call(
        _make_sel_body(PN, False),
        out_shape=[sds((NB, PN), F32), sds((NB, 128), F32)])(scoreT1, batchT)
    keptN1 = jnp.transpose(keptT1)
    gate116 = bcast16(jnp.tanh(score1))
    h1, kept16, sums1, mx1 = pl.pallas_call(
        _gate_pool_body, grid=(grid,),
        in_specs=[row128, row16, row16],
        out_specs=[row128, row16, accspec, accspec],
        out_shape=[sds((PN, H), F32), sds((PN, NB), F32),
                   sds((NB, H), F32), sds((NB, H), F32)])(h, gate116, keptN1)

    degp2 = seg16(kept16, srcp, dstp, z16)
    keptn = kept16[:, 0]
    deg2 = jnp.where(keptn > 0, degp2[0, :, 0] + degp2[1, :, 0] + 1.0, 1.0)
    dinv2 = jnp.where(deg2 > 0, deg2 ** -0.5, 0.0)
    dinv216 = bcast16(dinv2)
    hs2 = pl.pallas_call(
        _prep_body, grid=(grid,), in_specs=[row128, wspec, row16],
        out_specs=row128, out_shape=sds((PN, H), F32))(h1, W2, dinv216)
    featp2 = seg128(hs2, srcp, dstp, z128)
    h2, srel2, sroot2 = pl.pallas_call(
        _post2_body, grid=(grid,),
        in_specs=[pp128, row128, row16, bspec, w2spec, row16],
        out_specs=[row128, row16, row16],
        out_shape=[sds((PN, H), F32), sds((PN, NB), F32), sds((PN, NB), F32)])(
        featp2, hs2, dinv216, b2.reshape(1, H), wsc2, kept16)
    aggp2 = seg16(srel2, srcp, dstp, z16)

    score2 = aggp2[0, :, 0] + aggp2[1, :, 0] + bs2[0] + sroot2[:, 0]
    scoreT2 = jnp.broadcast_to(score2[None, :], (NB, PN))
    keptT2, cnt2 = pl.pallas_call(
        _make_sel_body(PN, True),
        out_shape=[sds((NB, PN), F32), sds((NB, 128), F32)])(
        scoreT2, batchT, keptT1)
    keptN2 = jnp.transpose(keptT2)
    gate216 = bcast16(jnp.tanh(score2))
    sums2, mx2 = pl.pallas_call(
        _pool2_body, grid=(grid,),
        in_specs=[row128, row16, row16],
        out_specs=[accspec, accspec],
        out_shape=[sds((NB, H), F32), sds((NB, H), F32)])(h2, gate216, keptN2)

    C = lin3_W.shape[1]
    logits, probs = pl.pallas_call(
        _head_body,
        out_shape=[sds((NB, C), F32), sds((NB, C), F32)])(
        sums1, mx1, cnt1, sums2, mx2, cnt2,
        lin1_W[:H], lin1_W[H:], lin1_b.reshape(1, H), lin2_W,
        lin2_b.reshape(1, lin2_W.shape[1]), lin3_W, lin3_b.reshape(1, C))
    return (logits, probs, label)


# speedup vs baseline: 1.1740x; 1.1740x over previous
import functools

import jax
import jax.numpy as jnp
from jax import lax
from jax.experimental import pallas as pl
from jax.experimental.pallas import tpu as pltpu
from jax.experimental.pallas import tpu_sc as plsc

RATIO = 0.5
NC, NS = 2, 16
NW = NC * NS
C_EDGE = 128
NB = 16
F32 = jnp.float32


def _sc_segsum(D, KI, PN, NH=1):
    rps = PN // NS
    NGH = KI // NH
    assert NGH * NH == KI and NGH % 2 == 0
    mesh = plsc.VectorSubcoreMesh(
        core_axis_name="c", subcore_axis_name="s", num_cores=NC, num_subcores=NS)

    @functools.partial(
        pl.kernel,
        out_type=jax.ShapeDtypeStruct((NC, PN, D), F32),
        mesh=mesh,
        compiler_params=pltpu.CompilerParams(use_tc_tiling_on_sc=False),
        scratch_types=[
            pltpu.VMEM((NGH, C_EDGE), jnp.int32),
            pltpu.VMEM((NGH, C_EDGE), jnp.int32),
            pltpu.VMEM((C_EDGE, D), F32),
            pltpu.VMEM((C_EDGE, D), F32),
            pltpu.VMEM_SHARED((PN, D), F32),
            pltpu.SemaphoreType.DMA,
            pltpu.SemaphoreType.DMA,
        ],
    )
    def k(table, srcp, dstp, zeros_h, out, srcv, dstv, rows0, rows1, acc,
          sem0, sem1):
        cid = lax.axis_index("c")
        sid = lax.axis_index("s")
        wid = sid * NC + cid
        pltpu.sync_copy(zeros_h.at[pl.ds(sid * rps, rps)],
                        acc.at[pl.ds(sid * rps, rps)])
        plsc.subcore_barrier()

        def body(p, carry):
            i = 2 * p
            pltpu.async_copy(table.at[srcv.at[i + 1]], rows1, sem1)
            pltpu.make_async_copy(table.at[pl.ds(0, C_EDGE)], rows0, sem0).wait()
            pltpu.sync_copy(rows0, acc.at[dstv.at[i]], add=True)

            @pl.when(i + 2 < NGH)
            def _():
                pltpu.async_copy(table.at[srcv.at[i + 2]], rows0, sem0)

            pltpu.make_async_copy(table.at[pl.ds(0, C_EDGE)], rows1, sem1).wait()
            pltpu.sync_copy(rows1, acc.at[dstv.at[i + 1]], add=True)
            return carry

        for hh in range(NH):
            pltpu.sync_copy(srcp.at[wid, pl.ds(hh * NGH, NGH)], srcv)
            pltpu.sync_copy(dstp.at[wid, pl.ds(hh * NGH, NGH)], dstv)
            pltpu.async_copy(table.at[srcv.at[0]], rows0, sem0)
            lax.fori_loop(0, NGH // 2, body, 0)

        plsc.subcore_barrier()
        pltpu.sync_copy(acc.at[pl.ds(sid * rps, rps)],
                        out.at[cid, pl.ds(sid * rps, rps)])

    return k


def _sc_histogram(KI, PN):
    D = NB
    rps = PN // NS
    mesh = plsc.VectorSubcoreMesh(
        core_axis_name="c", subcore_axis_name="s", num_cores=NC, num_subcores=NS)

    @functools.partial(
        pl.kernel,
        out_type=jax.ShapeDtypeStruct((NC, PN, D), F32),
        mesh=mesh,
        compiler_params=pltpu.CompilerParams(use_tc_tiling_on_sc=False),
        scratch_types=[
            pltpu.VMEM((KI, C_EDGE), jnp.int32),
            pltpu.VMEM((C_EDGE, D), F32),
            pltpu.VMEM_SHARED((PN, D), F32),
        ],
    )
    def k(ones_h, dstp, zeros_h, out, dstv, rows, acc):
        cid = lax.axis_index("c")
        sid = lax.axis_index("s")
        wid = sid * NC + cid
        pltpu.sync_copy(zeros_h.at[pl.ds(sid * rps, rps)],
                        acc.at[pl.ds(sid * rps, rps)])
        pltpu.sync_copy(dstp.at[wid], dstv)
        pltpu.sync_copy(ones_h.at[pl.ds(0, C_EDGE)], rows)
        plsc.subcore_barrier()

        def body(i, carry):
            pltpu.sync_copy(rows, acc.at[dstv.at[i]], add=True)
            return carry

        lax.fori_loop(0, KI, body, 0)
        plsc.subcore_barrier()
        pltpu.sync_copy(acc.at[pl.ds(sid * rps, rps)],
                        out.at[cid, pl.ds(sid * rps, rps)])

    return k


def _prep1_body(x_ref, w_ref, degp_ref, hs_ref):
    deg = degp_ref[0][:, :1] + degp_ref[1][:, :1] + 1.0
    dinv = lax.rsqrt(deg)
    hs_ref[...] = jnp.dot(x_ref[...], w_ref[...], preferred_element_type=F32) * dinv


def _prep2_body(h1_ref, w_ref, degp_ref, kept_ref, hs_ref):
    kept = kept_ref[...][:, :1]
    deg = jnp.where(kept > 0, degp_ref[0][:, :1] + degp_ref[1][:, :1] + 1.0, 1.0)
    dinv = lax.rsqrt(deg)
    hs_ref[...] = jnp.dot(h1_ref[...], w_ref[...], preferred_element_type=F32) * dinv


def _post1_body(pp_ref, hs_ref, degp_ref, b_ref, wsc_ref,
                h_ref, srel_ref, sroot_ref):
    deg = degp_ref[0][:, :1] + degp_ref[1][:, :1] + 1.0
    dinv = lax.rsqrt(deg)
    h = jnp.maximum(dinv * (pp_ref[0] + pp_ref[1] + hs_ref[...]) + b_ref[...], 0.0)
    h_ref[...] = h
    sb = jnp.dot(h, wsc_ref[...], preferred_element_type=F32)
    srel_ref[...] = jnp.broadcast_to(sb[:, :1], srel_ref.shape)
    sroot_ref[...] = jnp.broadcast_to(sb[:, 1:2], sroot_ref.shape)


def _post2_body(pp_ref, hs_ref, degp_ref, b_ref, wsc_ref, kept_ref,
                h_ref, srel_ref, sroot_ref):
    kept = kept_ref[...][:, :1]
    deg = jnp.where(kept > 0, degp_ref[0][:, :1] + degp_ref[1][:, :1] + 1.0, 1.0)
    dinv = lax.rsqrt(deg)
    h = jnp.maximum(dinv * (pp_ref[0] + pp_ref[1] + hs_ref[...]) + b_ref[...], 0.0)
    h = jnp.where(kept > 0, h, 0.0)
    h_ref[...] = h
    sb = jnp.dot(h, wsc_ref[...], preferred_element_type=F32)
    srel_ref[...] = jnp.broadcast_to(sb[:, :1], srel_ref.shape)
    sroot_ref[...] = jnp.broadcast_to(sb[:, 1:2], sroot_ref.shape)


def _make_sel_body(PN, with_elig):
    def select(aggT_ref, srootT_ref, batchT_ref, bs_ref, elig_ref,
               keptT_ref, cnt_ref):
        onehot = batchT_ref[...] == lax.broadcasted_iota(jnp.int32, (NB, PN), 0)
        if with_elig:
            onehot = onehot & (elig_ref[...] > 0)
        score = aggT_ref[0] + aggT_ref[1] + srootT_ref[...] + bs_ref[...]
        cnt = jnp.sum(onehot.astype(F32), axis=1, keepdims=True)
        kk = jnp.ceil(RATIO * cnt)
        u = lax.bitcast_convert_type(score, jnp.uint32)
        flip = jnp.where(u >> jnp.uint32(31) == jnp.uint32(1),
                         jnp.uint32(0xFFFFFFFF), jnp.uint32(0x80000000))
        key = u ^ flip

        def rbody(j, t):
            cand = t | jnp.left_shift(jnp.uint32(1), (31 - j).astype(jnp.uint32))
            c = jnp.sum(((key >= cand) & onehot).astype(F32), axis=1,
                        keepdims=True)
            return jnp.where(c >= kk, cand, t)

        V = lax.fori_loop(0, 32, rbody, jnp.zeros((NB, 1), jnp.uint32))
        gt = (key > V) & onehot
        eq = (key == V) & onehot
        m = kk - jnp.sum(gt.astype(F32), axis=1, keepdims=True)
        ridx = lax.broadcasted_iota(jnp.int32, (NB, PN), 1)

        def bbody(j, lohi):
            lo, hi = lohi
            mid = (lo + hi) // 2
            c = jnp.sum((eq & (ridx <= mid)).astype(F32), axis=1, keepdims=True)
            hit = c >= m
            return jnp.where(hit, lo, mid + 1), jnp.where(hit, mid, hi)

        _, J = lax.fori_loop(0, 14, bbody,
                             (jnp.zeros((NB, 1), jnp.int32),
                              jnp.full((NB, 1), PN - 1, jnp.int32)))
        kept = (gt | (eq & (ridx <= J))) & (kk >= 1.0)
        keptT_ref[...] = kept.astype(F32)
        ck = jnp.sum(kept.astype(F32), axis=1, keepdims=True)
        cnt_ref[...] = jnp.broadcast_to(ck, (NB, 128))

    if with_elig:
        def body(aggT_ref, srootT_ref, batchT_ref, bs_ref, elig_ref,
                 keptT_ref, cnt_ref):
            select(aggT_ref, srootT_ref, batchT_ref, bs_ref, elig_ref,
                   keptT_ref, cnt_ref)
    else:
        def body(aggT_ref, srootT_ref, batchT_ref, bs_ref,
                 keptT_ref, cnt_ref):
            select(aggT_ref, srootT_ref, batchT_ref, bs_ref, None,
                   keptT_ref, cnt_ref)
    return body


def _gate_pool_body(h_ref, aggp_ref, sroot_ref, bs_ref, keptn_ref,
                    h1_ref, kept16_ref, sums_ref, mx_ref):
    i = pl.program_id(0)
    score = aggp_ref[0] + aggp_ref[1] + sroot_ref[...] + bs_ref[...]
    kept_row = jnp.sum(keptn_ref[...], axis=1, keepdims=True)
    hg = h_ref[...] * (jnp.tanh(score[:, :1]) * kept_row)
    h1_ref[...] = hg
    kept16_ref[...] = jnp.broadcast_to(kept_row, kept16_ref.shape)
    part = lax.dot_general(keptn_ref[...], hg, (((0,), (0,)), ((), ())),
                           preferred_element_type=F32)
    neg = jnp.float32(-jnp.inf)
    rows = []
    for b in range(NB):
        maskb = keptn_ref[...][:, b:b + 1]
        rows.append(jnp.max(jnp.where(maskb > 0, hg, neg), axis=0, keepdims=True))
    tile_mx = jnp.concatenate(rows, axis=0)

    @pl.when(i == 0)
    def _():
        sums_ref[...] = part
        mx_ref[...] = tile_mx

    @pl.when(i > 0)
    def _():
        sums_ref[...] = sums_ref[...] + part
        mx_ref[...] = jnp.maximum(mx_ref[...], tile_mx)


def _pool2_body(h_ref, aggp_ref, sroot_ref, bs_ref, keptn_ref,
                sums_ref, mx_ref):
    i = pl.program_id(0)
    score = aggp_ref[0] + aggp_ref[1] + sroot_ref[...] + bs_ref[...]
    kept_row = jnp.sum(keptn_ref[...], axis=1, keepdims=True)
    hg = h_ref[...] * (jnp.tanh(score[:, :1]) * kept_row)
    part = lax.dot_general(keptn_ref[...], hg, (((0,), (0,)), ((), ())),
                           preferred_element_type=F32)
    neg = jnp.float32(-jnp.inf)
    rows = []
    for b in range(NB):
        maskb = keptn_ref[...][:, b:b + 1]
        rows.append(jnp.max(jnp.where(maskb > 0, hg, neg), axis=0, keepdims=True))
    tile_mx = jnp.concatenate(rows, axis=0)

    @pl.when(i == 0)
    def _():
        sums_ref[...] = part
        mx_ref[...] = tile_mx

    @pl.when(i > 0)
    def _():
        sums_ref[...] = sums_ref[...] + part
        mx_ref[...] = jnp.maximum(mx_ref[...], tile_mx)


def _head_body(sums1_ref, mx1_ref, cnt1_ref, sums2_ref, mx2_ref, cnt2_ref,
               w1a_ref, w1b_ref, b1_ref, w2_ref, b2_ref, w3_ref, b3_ref,
               logits_ref, probs_ref):
    c1 = cnt1_ref[...][:, :1]
    c2 = cnt2_ref[...][:, :1]
    mean1 = sums1_ref[...] / jnp.maximum(c1, 1.0)
    mean2 = sums2_ref[...] / jnp.maximum(c2, 1.0)
    mx1 = jnp.where(c1 > 0, mx1_ref[...], 0.0)
    mx2 = jnp.where(c2 > 0, mx2_ref[...], 0.0)
    zm = mean1 + mean2
    zx = mx1 + mx2
    z = jnp.maximum(
        jnp.dot(zm, w1a_ref[...], preferred_element_type=F32)
        + jnp.dot(zx, w1b_ref[...], preferred_element_type=F32)
        + b1_ref[...], 0.0)
    z = jnp.maximum(jnp.dot(z, w2_ref[...], preferred_element_type=F32)
                    + b2_ref[...], 0.0)
    logits = jnp.dot(z, w3_ref[...], preferred_element_type=F32) + b3_ref[...]
    logits_ref[...] = logits
    mmax = jnp.max(logits, axis=1, keepdims=True)
    e = jnp.exp(logits - mmax)
    probs_ref[...] = e / jnp.sum(e, axis=1, keepdims=True)


def kernel(x, edge_index, batch, label, W1, b1, Ws1_rel, bs1, Ws1_root, W2, b2,
           Ws2_rel, bs2, Ws2_root, lin1_W, lin1_b, lin2_W, lin2_b, lin3_W, lin3_b):
    N = x.shape[0]
    E = edge_index.shape[1]
    H = W1.shape[1]
    PN = ((N + 511) // 512) * 512
    KI = -(-E // (NW * C_EDGE))
    KI = KI + (KI % 2)
    Rb = 1024 if PN % 1024 == 0 else 512

    src = edge_index[0].astype(jnp.int32)
    dst = edge_index[1].astype(jnp.int32)
    pad_e = NW * KI * C_EDGE - E
    pad_dst = N + (jnp.arange(pad_e, dtype=jnp.int32) % (PN - N))
    srcp = jnp.concatenate([src, jnp.zeros((pad_e,), jnp.int32)]).reshape(NW, KI, C_EDGE)
    dstp = jnp.concatenate([dst, pad_dst]).reshape(NW, KI, C_EDGE)
    xp = jnp.pad(x, ((0, PN - N), (0, 0)))
    batch_pad = jnp.pad(batch.astype(jnp.int32), (0, PN - N), constant_values=NB)
    batch16 = jnp.broadcast_to(batch_pad[:, None], (PN, NB))
    batchT = jnp.broadcast_to(batch_pad[None, :], (NB, PN))
    ones16 = jnp.ones((PN, NB), F32)
    z16 = jnp.zeros((PN, NB), F32)
    z128 = jnp.zeros((PN, H), F32)
    wsc1 = jnp.concatenate([Ws1_rel, Ws1_root], axis=1)
    wsc2 = jnp.concatenate([Ws2_rel, Ws2_root], axis=1)

    seg16 = _sc_segsum(NB, KI, PN)
    seg128 = _sc_segsum(H, KI, PN, NH=2)

    full16 = pl.BlockSpec((2, Rb, NB), lambda i: (0, i, 0))
    row128 = pl.BlockSpec((Rb, H), lambda i: (i, 0))
    row16 = pl.BlockSpec((Rb, NB), lambda i: (i, 0))
    wspec = pl.BlockSpec((H, H), lambda i: (0, 0))
    w2spec = pl.BlockSpec((H, 2), lambda i: (0, 0))
    bspec = pl.BlockSpec((1, H), lambda i: (0, 0))
    b11spec = pl.BlockSpec((1, 1), lambda i: (0, 0))
    pp128 = pl.BlockSpec((2, Rb, H), lambda i: (0, i, 0))
    accspec = pl.BlockSpec((NB, H), lambda i: (0, 0))
    sds = jax.ShapeDtypeStruct
    grid = PN // Rb

    degp1 = _sc_histogram(KI, PN)(ones16, dstp, z16)
    hs1 = pl.pallas_call(
        _prep1_body, grid=(grid,), in_specs=[row128, wspec, full16],
        out_specs=row128, out_shape=sds((PN, H), F32))(xp, W1, degp1)
    featp1 = seg128(hs1, srcp, dstp, z128)
    h, srel16, sroot16 = pl.pallas_call(
        _post1_body, grid=(grid,),
        in_specs=[pp128, row128, full16, bspec, w2spec],
        out_specs=[row128, row16, row16],
        out_shape=[sds((PN, H), F32), sds((PN, NB), F32), sds((PN, NB), F32)])(
        featp1, hs1, degp1, b1.reshape(1, H), wsc1)
    aggp1 = seg16(srel16, srcp, dstp, z16)

    aggT_1 = jnp.stack([jnp.broadcast_to(aggp1[0, :, 0][None, :], (NB, PN)),
                        jnp.broadcast_to(aggp1[1, :, 0][None, :], (NB, PN))])
    srootT1 = jnp.broadcast_to(sroot16[:, 0][None, :], (NB, PN))
    keptT1, cnt1 = pl.pallas_call(
        _make_sel_body(PN, False),
        out_shape=[sds((NB, PN), F32), sds((NB, 128), F32)])(
        aggT_1, srootT1, batchT, bs1.reshape(1, 1))
    keptN1 = jnp.transpose(keptT1)
    h1, kept16, sums1, mx1 = pl.pallas_call(
        _gate_pool_body, grid=(grid,),
        in_specs=[row128, full16, row16, b11spec, row16],
        out_specs=[row128, row16, accspec, accspec],
        out_shape=[sds((PN, H), F32), sds((PN, NB), F32),
                   sds((NB, H), F32), sds((NB, H), F32)])(
        h, aggp1, sroot16, bs1.reshape(1, 1), keptN1)

    degp2 = seg16(kept16, srcp, dstp, z16)
    hs2 = pl.pallas_call(
        _prep2_body, grid=(grid,), in_specs=[row128, wspec, full16, row16],
        out_specs=row128, out_shape=sds((PN, H), F32))(h1, W2, degp2, kept16)
    featp2 = seg128(hs2, srcp, dstp, z128)
    h2, srel2, sroot2 = pl.pallas_call(
        _post2_body, grid=(grid,),
        in_specs=[pp128, row128, full16, bspec, w2spec, row16],
        out_specs=[row128, row16, row16],
        out_shape=[sds((PN, H), F32), sds((PN, NB), F32), sds((PN, NB), F32)])(
        featp2, hs2, degp2, b2.reshape(1, H), wsc2, kept16)
    aggp2 = seg16(srel2, srcp, dstp, z16)

    aggT_2 = jnp.stack([jnp.broadcast_to(aggp2[0, :, 0][None, :], (NB, PN)),
                        jnp.broadcast_to(aggp2[1, :, 0][None, :], (NB, PN))])
    srootT2 = jnp.broadcast_to(sroot2[:, 0][None, :], (NB, PN))
    keptT2, cnt2 = pl.pallas_call(
        _make_sel_body(PN, True),
        out_shape=[sds((NB, PN), F32), sds((NB, 128), F32)])(
        aggT_2, srootT2, batchT, bs2.reshape(1, 1), keptT1)
    keptN2 = jnp.transpose(keptT2)
    sums2, mx2 = pl.pallas_call(
        _pool2_body, grid=(grid,),
        in_specs=[row128, full16, row16, b11spec, row16],
        out_specs=[accspec, accspec],
        out_shape=[sds((NB, H), F32), sds((NB, H), F32)])(
        h2, aggp2, sroot2, bs2.reshape(1, 1), keptN2)

    C = lin3_W.shape[1]
    logits, probs = pl.pallas_call(
        _head_body,
        out_shape=[sds((NB, C), F32), sds((NB, C), F32)])(
        sums1, mx1, cnt1, sums2, mx2, cnt2,
        lin1_W[:H], lin1_W[H:], lin1_b.reshape(1, H), lin2_W,
        lin2_b.reshape(1, lin2_W.shape[1]), lin3_W, lin3_b.reshape(1, C))
    return (logits, probs, label)
